# Initial kernel scaffold; baseline (speedup 1.0000x reference)
import jax
import jax.numpy as jnp
from jax.experimental import pallas as pl


def kernel(idx, embed):
    raise NotImplementedError("write your pallas kernel here")


# speedup vs baseline: 5.9888x; 5.9888x over previous
import jax
import jax.numpy as jnp
import numpy as np
from jax import lax
from jax.experimental import pallas as pl
from jax.experimental.pallas import tpu as pltpu
from jax.experimental.pallas import tpu_sc as plsc

B = 16384
L = 200
VOCAB = 10
DIM = 10
N = B * L
NC = 2
NS = 16
NW = NC * NS
PER_W = N // NW
CHUNK = 512
NCHUNK = PER_W // CHUNK
LANES = 16


_PATS = np.array(
    [p // DIM for p in range(DIM * LANES)]
    + [p % DIM for p in range(DIM * LANES)],
    dtype=np.int32,
)


def _body(table_hbm, idx_hbm, pat_hbm, out_hbm, table_v, idx_v, out_v, pat_v, sem):
    wid = lax.axis_index("s") * NC + lax.axis_index("c")
    wbase = wid * PER_W

    pltpu.sync_copy(table_hbm, table_v)
    pltpu.sync_copy(pat_hbm, pat_v)

    rep = [pat_v[pl.ds(k * LANES, LANES)] for k in range(DIM)]
    dd = [pat_v[pl.ds(DIM * LANES + k * LANES, LANES)] for k in range(DIM)]

    def step(c, carry):
        base = wbase + c * CHUNK
        pltpu.sync_copy(idx_hbm.at[pl.ds(base, CHUNK)], idx_v)

        def group(g, carry2):
            gbase = g * LANES
            for k in range(DIM):
                pos = rep[k] + gbase
                ids = plsc.load_gather(idx_v, [pos])
                vals = plsc.load_gather(table_v, [ids, dd[k]])
                plsc.store_scatter(out_v, [pos, dd[k]], vals)
            return carry2

        lax.fori_loop(0, CHUNK // LANES, group, 0)
        pltpu.sync_copy(out_v, out_hbm.at[pl.ds(base, CHUNK)])
        return carry

    lax.fori_loop(0, NCHUNK, step, 0)


@jax.jit
def _lookup(embed, idxf):
    mesh = plsc.VectorSubcoreMesh(core_axis_name="c", subcore_axis_name="s")
    return pl.kernel(
        _body,
        out_type=jax.ShapeDtypeStruct((N, DIM), jnp.float32),
        mesh=mesh,
        compiler_params=pltpu.CompilerParams(needs_layout_passes=False),
        scratch_types=[
            pltpu.VMEM((VOCAB, DIM), jnp.float32),
            pltpu.VMEM((CHUNK,), jnp.int32),
            pltpu.VMEM((CHUNK, DIM), jnp.float32),
            pltpu.VMEM((2 * DIM * LANES,), jnp.int32),
            pltpu.SemaphoreType.DMA,
        ],
    )(embed, idxf, jnp.asarray(_PATS))


def kernel(idx, embed):
    out = _lookup(embed, idx.reshape(-1))
    return out.reshape(B, L, DIM)


# speedup vs baseline: 9.6227x; 1.6068x over previous
import jax
import jax.numpy as jnp
import numpy as np
from jax import lax
from jax.experimental import pallas as pl
from jax.experimental.pallas import tpu as pltpu
from jax.experimental.pallas import tpu_sc as plsc

B = 16384
L = 200
VOCAB = 10
DIM = 10
N = B * L
NC = 2
NS = 16
NW = NC * NS
ROWS_W = B // NW
CB = 2
CHUNK = CB * L
NCH = ROWS_W // CB
NCHH = NCH // 2
LANES = 16
GCOLS = tuple(range(0, L - LANES + 1, LANES)) + (L - LANES,)

_PATS = np.array(
    [p // DIM for p in range(DIM * LANES)]
    + [p % DIM for p in range(DIM * LANES)],
    dtype=np.int32,
)

_TAKE_DNUMS = lax.GatherDimensionNumbers(
    offset_dims=(), collapsed_slice_dims=(0,), start_index_map=(0,)
)


def _permute(vec, pat):
    return lax.gather(
        vec,
        pat.reshape(LANES, 1),
        dimension_numbers=_TAKE_DNUMS,
        slice_sizes=(1,),
        mode=lax.GatherScatterMode.PROMISE_IN_BOUNDS,
    )


def _body(table_hbm, idx_hbm, pat_hbm, out_hbm,
          table_v, pat_v, ib0, ib1, ob0, ob1,
          tsem, is0, is1, os0, os1):
    wid = lax.axis_index("s") * NC + lax.axis_index("c")
    wrow = wid * ROWS_W

    pltpu.sync_copy(table_hbm, table_v)
    pltpu.sync_copy(pat_hbm, pat_v)

    rep = [pat_v[pl.ds(k * LANES, LANES)] for k in range(DIM)]
    dd = [pat_v[pl.ds(DIM * LANES + k * LANES, LANES)] for k in range(DIM)]

    ibufs = (ib0, ib1)
    obufs = (ob0, ob1)
    isems = (is0, is1)
    osems = (os0, os1)

    pltpu.async_copy(idx_hbm.at[pl.ds(wrow, CB)], ib0, is0)
    pltpu.async_copy(idx_hbm.at[pl.ds(wrow + CB, CB)], ib1, is1)

    def outer(cc, carry):
        for b in range(2):
            c = cc * 2 + b
            ibuf, obuf = ibufs[b], obufs[b]
            isem, osem = isems[b], osems[b]
            rowbase = wrow + c * CB

            pltpu.make_async_copy(idx_hbm.at[pl.ds(wrow, CB)], ibuf, isem).wait()

            @pl.when(cc > 0)
            def _():
                pltpu.make_async_copy(
                    obuf, out_hbm.at[pl.ds(0, CHUNK)], osem
                ).wait()

            for r in range(CB):
                for gcol in GCOLS:
                    ids16 = ibuf[r, pl.ds(gcol, LANES)]
                    dst = obuf.at[pl.ds(r * L + gcol, LANES)]
                    for k in range(DIM):
                        idsr = _permute(ids16, rep[k])
                        vals = plsc.load_gather(table_v, [idsr, dd[k]])
                        plsc.store_scatter(dst, [rep[k], dd[k]], vals)

            pltpu.async_copy(
                obuf, out_hbm.at[pl.ds(rowbase * L, CHUNK)], osem
            )

            @pl.when(cc < NCHH - 1)
            def _():
                pltpu.async_copy(
                    idx_hbm.at[pl.ds(rowbase + 2 * CB, CB)], ibuf, isem
                )
        return carry

    lax.fori_loop(0, NCHH, outer, 0)

    pltpu.make_async_copy(ob0, out_hbm.at[pl.ds(0, CHUNK)], os0).wait()
    pltpu.make_async_copy(ob1, out_hbm.at[pl.ds(0, CHUNK)], os1).wait()


@jax.jit
def _lookup(embed, idx2d):
    mesh = plsc.VectorSubcoreMesh(core_axis_name="c", subcore_axis_name="s")
    return pl.kernel(
        _body,
        out_type=jax.ShapeDtypeStruct((N, DIM), jnp.float32),
        mesh=mesh,
        compiler_params=pltpu.CompilerParams(needs_layout_passes=False),
        scratch_types=[
            pltpu.VMEM((VOCAB, DIM), jnp.float32),
            pltpu.VMEM((2 * DIM * LANES,), jnp.int32),
            pltpu.VMEM((CB, L), jnp.int32),
            pltpu.VMEM((CB, L), jnp.int32),
            pltpu.VMEM((CHUNK, DIM), jnp.float32),
            pltpu.VMEM((CHUNK, DIM), jnp.float32),
            pltpu.SemaphoreType.DMA,
            pltpu.SemaphoreType.DMA,
            pltpu.SemaphoreType.DMA,
            pltpu.SemaphoreType.DMA,
            pltpu.SemaphoreType.DMA,
        ],
    )(embed, idx2d, jnp.asarray(_PATS))


def kernel(idx, embed):
    out = _lookup(embed, idx)
    return out.reshape(B, L, DIM)


# speedup vs baseline: 14.3926x; 1.4957x over previous
import jax
import jax.numpy as jnp
from jax import lax
from jax.experimental import pallas as pl
from jax.experimental.pallas import tpu as pltpu
from jax.experimental.pallas import tpu_sc as plsc

B = 16384
L = 200
VOCAB = 10
DIM = 10
NC = 2
NS = 16
NW = NC * NS
BW = B // NW
BBLK = 128
NBLK = BW // BBLK
LC = 8
NLC = L // LC
LANES = 16
NG = BBLK // LANES
TROW = 128


def _body(tabf_hbm, sidx_hbm, out_hbm,
          tabf_v, ix0, ix1, ob0, ob1,
          is0, is1, os0, os1):
    wid = lax.axis_index("s") * NC + lax.axis_index("c")
    bw0 = wid * BW

    pltpu.sync_copy(tabf_hbm, tabf_v)

    iota16 = lax.iota(jnp.int32, LANES)

    ixbufs, ixsems = (ix0, ix1), (is0, is1)
    obufs, osems = (ob0, ob1), (os0, os1)

    pltpu.async_copy(sidx_hbm.at[pl.ds(bw0, BBLK)], ix0, is0)
    pltpu.async_copy(sidx_hbm.at[pl.ds(bw0 + BBLK, BBLK)], ix1, is1)

    def compute_chunk(ibuf, obuf, l0):
        def lqloop(lq, c):
            lspl = iota16 * 0 + (l0 + lq)
            for g in range(NG):
                pos = iota16 + g * LANES
                sids = plsc.load_gather(ibuf, [pos, lspl])
                for d in range(DIM):
                    vals = plsc.load_gather(tabf_v, [sids + d])
                    obuf[d, lq, pl.ds(g * LANES, LANES)] = vals
            return c

        lax.fori_loop(0, LC, lqloop, 0)

    def wait_out(obuf, osem):
        pltpu.make_async_copy(
            obuf, out_hbm.at[:, pl.ds(0, LC), pl.ds(0, BBLK)], osem
        ).wait()

    def block(blk, carry):
        for ip in range(2):
            @pl.when(lax.rem(blk, 2) == ip)
            def _(ip=ip):
                ibuf, isem = ixbufs[ip], ixsems[ip]
                pltpu.make_async_copy(
                    sidx_hbm.at[pl.ds(0, BBLK)], ibuf, isem
                ).wait()

                def chunkloop(lc, c2):
                    for op in range(2):
                        @pl.when(lax.rem(blk + lc, 2) == op)
                        def _(op=op):
                            obuf, osem = obufs[op], osems[op]

                            @pl.when(blk * NLC + lc >= 2)
                            def _():
                                wait_out(obuf, osem)

                            l0 = lc * LC
                            compute_chunk(ibuf, obuf, l0)
                            pltpu.async_copy(
                                obuf,
                                out_hbm.at[
                                    :, pl.ds(l0, LC),
                                    pl.ds(bw0 + blk * BBLK, BBLK),
                                ],
                                osem,
                            )
                    return c2

                lax.fori_loop(0, NLC, chunkloop, 0)

                @pl.when(blk < NBLK - 2)
                def _():
                    pltpu.async_copy(
                        sidx_hbm.at[pl.ds(bw0 + (blk + 2) * BBLK, BBLK)],
                        ibuf, isem,
                    )
        return carry

    lax.fori_loop(0, NBLK, block, 0)

    wait_out(ob0, os0)
    wait_out(ob1, os1)


@jax.jit
def _lookup(tabf, sidx):
    mesh = plsc.VectorSubcoreMesh(core_axis_name="c", subcore_axis_name="s")
    return pl.kernel(
        _body,
        out_type=jax.ShapeDtypeStruct((DIM, L, B), jnp.float32),
        mesh=mesh,
        compiler_params=pltpu.CompilerParams(needs_layout_passes=False),
        scratch_types=[
            pltpu.VMEM((VOCAB * TROW,), jnp.float32),
            pltpu.VMEM((BBLK, L), jnp.int32),
            pltpu.VMEM((BBLK, L), jnp.int32),
            pltpu.VMEM((DIM, LC, BBLK), jnp.float32),
            pltpu.VMEM((DIM, LC, BBLK), jnp.float32),
            pltpu.SemaphoreType.DMA,
            pltpu.SemaphoreType.DMA,
            pltpu.SemaphoreType.DMA,
            pltpu.SemaphoreType.DMA,
        ],
    )(tabf, sidx)


def kernel(idx, embed):
    sidx = jnp.minimum(idx, VOCAB - 1) * TROW
    tabf = jnp.pad(embed, ((0, 0), (0, TROW - DIM))).reshape(-1)
    out_t = _lookup(tabf, sidx)
    return out_t.transpose(2, 1, 0)


# speedup vs baseline: 23.7433x; 1.6497x over previous
import jax
import jax.numpy as jnp
from jax import lax
from jax.experimental import pallas as pl
from jax.experimental.pallas import tpu as pltpu
from jax.experimental.pallas import tpu_sc as plsc

B = 16384
L = 200
VOCAB = 10
DIM = 10
NC = 2
NS = 16
NW = NC * NS
BW = B // NW
BBLK = 128
NBLK = BW // BBLK
LC = 8
NLC = L // LC
LANES = 16
NG = BBLK // LANES
TROW = 128


def _body(tabf_hbm, sidx_hbm, out_hbm,
          tabf_v, ix0, ix1, ob0, ob1,
          is0, is1, os0, os1):
    wid = lax.axis_index("s") * NC + lax.axis_index("c")
    bw0 = wid * BW

    pltpu.sync_copy(tabf_hbm, tabf_v)

    iota16 = lax.iota(jnp.int32, LANES)

    ixbufs, ixsems = (ix0, ix1), (is0, is1)
    obufs, osems = (ob0, ob1), (os0, os1)

    pltpu.async_copy(sidx_hbm.at[pl.ds(bw0, BBLK)], ix0, is0)
    pltpu.async_copy(sidx_hbm.at[pl.ds(bw0 + BBLK, BBLK)], ix1, is1)

    def compute_chunk(ibuf, obuf, l0):
        def lqloop(lq, c):
            lspl = iota16 * 0 + (l0 + lq)
            sids_g = []
            for g in range(NG):
                pos = iota16 + g * LANES
                sids_g.append(plsc.load_gather(ibuf, [pos, lspl]))
            for g in range(NG):
                vals = [
                    plsc.load_gather(tabf_v, [sids_g[g] + d]) for d in range(DIM)
                ]
                for d in range(DIM):
                    obuf[d, lq, pl.ds(g * LANES, LANES)] = vals[d]
            return c

        lax.fori_loop(0, LC, lqloop, 0)

    def wait_out(obuf, osem):
        pltpu.make_async_copy(
            obuf, out_hbm.at[:, pl.ds(0, LC), pl.ds(0, BBLK)], osem
        ).wait()

    def block(blk, carry):
        for ip in range(2):
            @pl.when(lax.rem(blk, 2) == ip)
            def _(ip=ip):
                ibuf, isem = ixbufs[ip], ixsems[ip]
                pltpu.make_async_copy(
                    sidx_hbm.at[pl.ds(0, BBLK)], ibuf, isem
                ).wait()

                def chunkloop(lc, c2):
                    for op in range(2):
                        @pl.when(lax.rem(blk + lc, 2) == op)
                        def _(op=op):
                            obuf, osem = obufs[op], osems[op]

                            @pl.when(blk * NLC + lc >= 2)
                            def _():
                                wait_out(obuf, osem)

                            l0 = lc * LC
                            compute_chunk(ibuf, obuf, l0)
                            pltpu.async_copy(
                                obuf,
                                out_hbm.at[
                                    :, pl.ds(l0, LC),
                                    pl.ds(bw0 + blk * BBLK, BBLK),
                                ],
                                osem,
                            )
                    return c2

                lax.fori_loop(0, NLC, chunkloop, 0)

                @pl.when(blk < NBLK - 2)
                def _():
                    pltpu.async_copy(
                        sidx_hbm.at[pl.ds(bw0 + (blk + 2) * BBLK, BBLK)],
                        ibuf, isem,
                    )
        return carry

    lax.fori_loop(0, NBLK, block, 0)

    wait_out(ob0, os0)
    wait_out(ob1, os1)


@jax.jit
def _lookup(tabf, sidx):
    mesh = plsc.VectorSubcoreMesh(core_axis_name="c", subcore_axis_name="s")
    return pl.kernel(
        _body,
        out_type=jax.ShapeDtypeStruct((DIM, L, B), jnp.float32),
        mesh=mesh,
        compiler_params=pltpu.CompilerParams(needs_layout_passes=False),
        scratch_types=[
            pltpu.VMEM((VOCAB * TROW,), jnp.float32),
            pltpu.VMEM((BBLK, L), jnp.int32),
            pltpu.VMEM((BBLK, L), jnp.int32),
            pltpu.VMEM((DIM, LC, BBLK), jnp.float32),
            pltpu.VMEM((DIM, LC, BBLK), jnp.float32),
            pltpu.SemaphoreType.DMA,
            pltpu.SemaphoreType.DMA,
            pltpu.SemaphoreType.DMA,
            pltpu.SemaphoreType.DMA,
        ],
    )(tabf, sidx)


def kernel(idx, embed):
    sidx = jnp.minimum(idx, VOCAB - 1) * TROW
    tabf = jnp.pad(embed, ((0, 0), (0, TROW - DIM))).reshape(-1)
    out_t = _lookup(tabf, sidx)
    return out_t.transpose(2, 1, 0)


# speedup vs baseline: 118.2581x; 4.9807x over previous
import jax
import jax.numpy as jnp
from jax import lax
from jax.experimental import pallas as pl
from jax.experimental.pallas import tpu as pltpu
from jax.experimental.pallas import tpu_sc as plsc

B = 16384
L = 200
VOCAB = 10
DIM = 10
NC = 2
NS = 16
NW = NC * NS
BW = B // NW
BBLK = 128
NBLK = BW // BBLK
LC = 8
NLC = L // LC
LANES = 16
NG = BBLK // LANES
TROW = LANES


def _body(tabf_hbm, sidx_hbm, out_hbm,
          tabf_v, ix0, ix1, ob0, ob1,
          is0, is1, os0, os1):
    wid = lax.axis_index("s") * NC + lax.axis_index("c")
    bw0 = wid * BW

    pltpu.sync_copy(tabf_hbm, tabf_v)

    iota16 = lax.iota(jnp.int32, LANES)

    ixbufs, ixsems = (ix0, ix1), (is0, is1)
    obufs, osems = (ob0, ob1), (os0, os1)

    pltpu.async_copy(sidx_hbm.at[:, pl.ds(bw0, BBLK)], ix0, is0)
    pltpu.async_copy(sidx_hbm.at[:, pl.ds(bw0 + BBLK, BBLK)], ix1, is1)

    def compute_chunk(ibuf, obuf, l0):
        def lqloop(lq, c):
            l = l0 + lq
            ids_g = [ibuf[l, pl.ds(g * LANES, LANES)] for g in range(NG)]
            for g in range(NG):
                vals = [
                    plsc.load_gather(tabf_v, [ids_g[g] + d * TROW])
                    for d in range(DIM)
                ]
                for d in range(DIM):
                    obuf[d, lq, pl.ds(g * LANES, LANES)] = vals[d]
            return c

        lax.fori_loop(0, LC, lqloop, 0)

    def wait_out(obuf, osem):
        pltpu.make_async_copy(
            obuf, out_hbm.at[:, pl.ds(0, LC), pl.ds(0, BBLK)], osem
        ).wait()

    def block(blk, carry):
        for ip in range(2):
            @pl.when(lax.rem(blk, 2) == ip)
            def _(ip=ip):
                ibuf, isem = ixbufs[ip], ixsems[ip]
                pltpu.make_async_copy(
                    sidx_hbm.at[:, pl.ds(0, BBLK)], ibuf, isem
                ).wait()

                def chunkloop(lc, c2):
                    for op in range(2):
                        @pl.when(lax.rem(blk + lc, 2) == op)
                        def _(op=op):
                            obuf, osem = obufs[op], osems[op]

                            @pl.when(blk * NLC + lc >= 2)
                            def _():
                                wait_out(obuf, osem)

                            l0 = lc * LC
                            compute_chunk(ibuf, obuf, l0)
                            pltpu.async_copy(
                                obuf,
                                out_hbm.at[
                                    :, pl.ds(l0, LC),
                                    pl.ds(bw0 + blk * BBLK, BBLK),
                                ],
                                osem,
                            )
                    return c2

                lax.fori_loop(0, NLC, chunkloop, 0)

                @pl.when(blk < NBLK - 2)
                def _():
                    pltpu.async_copy(
                        sidx_hbm.at[:, pl.ds(bw0 + (blk + 2) * BBLK, BBLK)],
                        ibuf, isem,
                    )
        return carry

    lax.fori_loop(0, NBLK, block, 0)

    wait_out(ob0, os0)
    wait_out(ob1, os1)


@jax.jit
def _lookup(tabf, sidx):
    mesh = plsc.VectorSubcoreMesh(core_axis_name="c", subcore_axis_name="s")
    return pl.kernel(
        _body,
        out_type=jax.ShapeDtypeStruct((DIM, L, B), jnp.float32),
        mesh=mesh,
        compiler_params=pltpu.CompilerParams(needs_layout_passes=False),
        scratch_types=[
            pltpu.VMEM((DIM * TROW,), jnp.float32),
            pltpu.VMEM((L, BBLK), jnp.int32),
            pltpu.VMEM((L, BBLK), jnp.int32),
            pltpu.VMEM((DIM, LC, BBLK), jnp.float32),
            pltpu.VMEM((DIM, LC, BBLK), jnp.float32),
            pltpu.SemaphoreType.DMA,
            pltpu.SemaphoreType.DMA,
            pltpu.SemaphoreType.DMA,
            pltpu.SemaphoreType.DMA,
        ],
    )(tabf, sidx)


def kernel(idx, embed):
    sidx = jnp.minimum(idx, VOCAB - 1).T
    tabf = jnp.pad(embed.T, ((0, 0), (0, TROW - VOCAB))).reshape(-1)
    out_t = _lookup(tabf, sidx)
    return out_t.transpose(2, 1, 0)
